# baseline (speedup 1.0000x reference)
import jax
import jax.numpy as jnp
from jax.experimental import pallas as pl
from jax.experimental.pallas import tpu as pltpu

T = 16
N = 512
HIDDEN = 64
MEDIATE = 32
SOCIAL = 128
OUT_DIM = 2
NSIZE = 4
GRID = 1.0

_BINS = (-1.0, 0.0, 1.0)


def _social_lstm_body(X_ref, C0_ref, C1_ref, MC_ref, Y_ref, h0_ref, c0_ref,
                      WinT_ref, bin_ref, WsocT_ref, bsoc_ref,
                      WihT_ref, Whh_ref, bih_ref, bhh_ref,
                      Wout_ref, bout_ref, tp_ref, out_ref):
    tpred = tp_ref[0, 0]

    def frame(t, carry):
        h, c = carry
        mcol = MC_ref[pl.ds(t, 1), :, :].reshape(N, 1)
        mb = (mcol != 0.0).astype(jnp.float32)
        hm = h * mb

        xc = X_ref[pl.ds(t, 1), :, 2:4].reshape(N, 2)
        x0c = xc[:, 0:1]
        x1c = xc[:, 1:2]
        x0r = C0_ref[pl.ds(t, 1), :]
        x1r = C1_ref[pl.ds(t, 1), :]
        d0 = x0r - x0c
        d1 = x1r - x1c

        r_ind = ((d0 > -2.0) & (d0 <= -1.0),
                 (d0 > -1.0) & (d0 < 1.0),
                 (d0 >= 1.0) & (d0 < 2.0))
        c_ind = ((d1 > -2.0) & (d1 <= -1.0),
                 (d1 > -1.0) & (d1 < 1.0),
                 (d1 >= 1.0) & (d1 < 2.0))

        masks = {}
        for ai, bi in ((0, 0), (0, 1), (0, 2), (1, 0), (1, 1)):
            masks[(ai, bi)] = (r_ind[ai] & c_ind[bi]).astype(jnp.float32)

        zero_blk = jnp.zeros((N, HIDDEN), jnp.float32)
        Hs = [zero_blk] * (NSIZE * NSIZE)
        for ai in range(3):
            for bi in range(3):
                if (ai, bi) in masks:
                    Hab = jnp.dot(masks[(ai, bi)], hm,
                                  preferred_element_type=jnp.float32)
                else:
                    Hab = jax.lax.dot_general(
                        masks[(2 - ai, 2 - bi)], hm,
                        (((0,), (0,)), ((), ())),
                        preferred_element_type=jnp.float32)
                if ai == 1 and bi == 1:
                    Hab = Hab - hm
                Hs[(ai + 1) * NSIZE + (bi + 1)] = Hab
        Hfull = jnp.concatenate(Hs, axis=1)

        epre = jnp.dot(Hfull, WsocT_ref[...], preferred_element_type=jnp.float32)
        e = jax.nn.relu(epre * mb + bsoc_ref[...])

        r = jax.nn.relu(jnp.dot(xc, WinT_ref[...],
                                preferred_element_type=jnp.float32)
                        + bin_ref[...])

        concat = jnp.concatenate([r, e], axis=1)
        gates = ((jnp.dot(concat, WihT_ref[...], preferred_element_type=jnp.float32)
                  + bih_ref[...])
                 + jnp.dot(h, Whh_ref[...], preferred_element_type=jnp.float32)
                 + bhh_ref[...])
        gi = gates[:, 0 * HIDDEN:1 * HIDDEN]
        gf = gates[:, 1 * HIDDEN:2 * HIDDEN]
        gg = gates[:, 2 * HIDDEN:3 * HIDDEN]
        go = gates[:, 3 * HIDDEN:4 * HIDDEN]
        c2 = jax.nn.sigmoid(gf) * c + jax.nn.sigmoid(gi) * jnp.tanh(gg)
        h2 = jax.nn.sigmoid(go) * jnp.tanh(c2)

        o = (jnp.dot(h2, Wout_ref[...], preferred_element_type=jnp.float32)
             + bout_ref[...]) * mcol

        i3 = jnp.maximum(t - 3, 0)
        m3 = MC_ref[pl.ds(i3, 1), :, :].reshape(N, 1)
        cond = (mcol != 0.0) & (m3 == 0.0) & (t > 3)
        yv = Y_ref[pl.ds(t, 1), :, :].reshape(N, OUT_DIM)
        o = jnp.where(cond, yv, o)

        active = t <= tpred
        o = jnp.where(active, o, 0.0)
        h = jnp.where(active, h2, h)
        c = jnp.where(active, c2, c)
        out_ref[pl.ds(t, 1), :, :] = o.reshape(1, N, OUT_DIM)
        return (h, c)

    jax.lax.fori_loop(0, T, frame, (h0_ref[...], c0_ref[...]))


def kernel(X, part_masks, all_h_t, all_c_t, Y, W_in, b_in, W_soc, b_soc,
           W_ih, W_hh, b_ih, b_hh, W_out, b_out, T_obs, T_pred):
    C0 = X[:, :, 2]
    C1 = X[:, :, 3]
    MC = part_masks[:, 0, :, None]

    WsocT = W_soc.T
    WinT = W_in.T
    WihT = W_ih.T
    Whh = W_hh.T
    Wout = W_out.T
    tp = jnp.asarray(T_pred, jnp.int32).reshape(1, 1)

    in_specs = [pl.BlockSpec(memory_space=pltpu.VMEM)] * 17 + [
        pl.BlockSpec(memory_space=pltpu.SMEM)]

    return pl.pallas_call(
        _social_lstm_body,
        out_shape=jax.ShapeDtypeStruct((T, N, OUT_DIM), jnp.float32),
        in_specs=in_specs,
        out_specs=pl.BlockSpec(memory_space=pltpu.VMEM),
    )(X, C0, C1, MC, Y, all_h_t, all_c_t,
      WinT, b_in[None, :], WsocT, b_soc[None, :],
      WihT, Whh, b_ih[None, :], b_hh[None, :],
      Wout, b_out[None, :], tp)


# speedup vs baseline: 1.0589x; 1.0589x over previous
import jax
import jax.numpy as jnp
from jax.experimental import pallas as pl
from jax.experimental.pallas import tpu as pltpu

T = 16
N = 512
HIDDEN = 64
MEDIATE = 32
SOCIAL = 128
OUT_DIM = 2
NSIZE = 4
GRID = 1.0

_BINS = (-1.0, 0.0, 1.0)


def _social_lstm_body(X_ref, C0_ref, C1_ref, MC_ref, Y_ref, h0_ref, c0_ref,
                      WinT_ref, bin_ref, WsocT_ref, bsoc_ref,
                      WihT_ref, Whh_ref, bih_ref, bhh_ref,
                      Wout_ref, bout_ref, tp_ref, out_ref):
    tpred = tp_ref[0, 0]

    def frame(t, carry):
        h, c = carry
        mcol = MC_ref[pl.ds(t, 1), :, :].reshape(N, 1)
        mb = (mcol != 0.0).astype(jnp.float32)
        hm = h * mb

        xc = X_ref[pl.ds(t, 1), :, 2:4].reshape(N, 2)
        x0c = xc[:, 0:1]
        x1c = xc[:, 1:2]
        x0r = C0_ref[pl.ds(t, 1), :]
        x1r = C1_ref[pl.ds(t, 1), :]
        d0 = x0r - x0c
        d1 = x1r - x1c

        r_ind = ((d0 > -2.0) & (d0 <= -1.0),
                 (d0 > -1.0) & (d0 < 1.0),
                 (d0 >= 1.0) & (d0 < 2.0))
        c_ind = ((d1 > -2.0) & (d1 <= -1.0),
                 (d1 > -1.0) & (d1 < 1.0),
                 (d1 >= 1.0) & (d1 < 2.0))

        zero_blk = jnp.zeros((N, HIDDEN), jnp.float32)
        Hs = [zero_blk] * (NSIZE * NSIZE)
        for ai in range(3):
            for bi in range(3):
                M = (r_ind[ai] & c_ind[bi]).astype(jnp.float32)
                Hab = jnp.dot(M, hm, preferred_element_type=jnp.float32)
                if ai == 1 and bi == 1:
                    Hab = Hab - hm
                Hs[(ai + 1) * NSIZE + (bi + 1)] = Hab
        Hfull = jnp.concatenate(Hs, axis=1)

        epre = jnp.dot(Hfull, WsocT_ref[...], preferred_element_type=jnp.float32)
        e = jax.nn.relu(epre * mb + bsoc_ref[...])

        r = jax.nn.relu(jnp.dot(xc, WinT_ref[...],
                                preferred_element_type=jnp.float32)
                        + bin_ref[...])

        concat = jnp.concatenate([r, e], axis=1)
        gates = ((jnp.dot(concat, WihT_ref[...], preferred_element_type=jnp.float32)
                  + bih_ref[...])
                 + jnp.dot(h, Whh_ref[...], preferred_element_type=jnp.float32)
                 + bhh_ref[...])
        gi = gates[:, 0 * HIDDEN:1 * HIDDEN]
        gf = gates[:, 1 * HIDDEN:2 * HIDDEN]
        gg = gates[:, 2 * HIDDEN:3 * HIDDEN]
        go = gates[:, 3 * HIDDEN:4 * HIDDEN]
        c2 = jax.nn.sigmoid(gf) * c + jax.nn.sigmoid(gi) * jnp.tanh(gg)
        h2 = jax.nn.sigmoid(go) * jnp.tanh(c2)

        o = (jnp.dot(h2, Wout_ref[...], preferred_element_type=jnp.float32)
             + bout_ref[...]) * mcol

        i3 = jnp.maximum(t - 3, 0)
        m3 = MC_ref[pl.ds(i3, 1), :, :].reshape(N, 1)
        cond = (mcol != 0.0) & (m3 == 0.0) & (t > 3)
        yv = Y_ref[pl.ds(t, 1), :, :].reshape(N, OUT_DIM)
        o = jnp.where(cond, yv, o)

        active = t <= tpred
        o = jnp.where(active, o, 0.0)
        h = jnp.where(active, h2, h)
        c = jnp.where(active, c2, c)
        out_ref[pl.ds(t, 1), :, :] = o.reshape(1, N, OUT_DIM)
        return (h, c)

    jax.lax.fori_loop(0, T, frame, (h0_ref[...], c0_ref[...]))


def kernel(X, part_masks, all_h_t, all_c_t, Y, W_in, b_in, W_soc, b_soc,
           W_ih, W_hh, b_ih, b_hh, W_out, b_out, T_obs, T_pred):
    C0 = X[:, :, 2]
    C1 = X[:, :, 3]
    MC = part_masks[:, 0, :, None]

    WsocT = W_soc.T
    WinT = W_in.T
    WihT = W_ih.T
    Whh = W_hh.T
    Wout = W_out.T
    tp = jnp.asarray(T_pred, jnp.int32).reshape(1, 1)

    in_specs = [pl.BlockSpec(memory_space=pltpu.VMEM)] * 17 + [
        pl.BlockSpec(memory_space=pltpu.SMEM)]

    return pl.pallas_call(
        _social_lstm_body,
        out_shape=jax.ShapeDtypeStruct((T, N, OUT_DIM), jnp.float32),
        in_specs=in_specs,
        out_specs=pl.BlockSpec(memory_space=pltpu.VMEM),
    )(X, C0, C1, MC, Y, all_h_t, all_c_t,
      WinT, b_in[None, :], WsocT, b_soc[None, :],
      WihT, Whh, b_ih[None, :], b_hh[None, :],
      Wout, b_out[None, :], tp)


# speedup vs baseline: 1.2329x; 1.1643x over previous
import jax
import jax.numpy as jnp
from jax.experimental import pallas as pl
from jax.experimental.pallas import tpu as pltpu

T = 16
N = 512
HIDDEN = 64
MEDIATE = 32
SOCIAL = 128
OUT_DIM = 2
NSIZE = 4
GRID = 1.0

_BINS = (-1.0, 0.0, 1.0)


def _social_lstm_body(X_ref, C0_ref, C1_ref, MC_ref, Y_ref, h0_ref, c0_ref,
                      WinT_ref, bin_ref, WsocT_ref, bsoc_ref,
                      WihT_ref, Whh_ref, bih_ref, bhh_ref,
                      Wout_ref, bout_ref, tp_ref, out_ref):
    tpred = tp_ref[0, 0]
    out_ref[...] = jnp.zeros((T, N, OUT_DIM), jnp.float32)

    def frame(t, carry):
        h, c = carry
        mcol = MC_ref[pl.ds(t, 1), :, :].reshape(N, 1)
        mb = (mcol != 0.0).astype(jnp.float32)
        hm = h * mb

        xc = X_ref[pl.ds(t, 1), :, 2:4].reshape(N, 2)
        x0c = xc[:, 0:1]
        x1c = xc[:, 1:2]
        x0r = C0_ref[pl.ds(t, 1), :]
        x1r = C1_ref[pl.ds(t, 1), :]
        d0 = x0r - x0c
        d1 = x1r - x1c

        r_ind = ((d0 > -2.0) & (d0 <= -1.0),
                 (d0 > -1.0) & (d0 < 1.0),
                 (d0 >= 1.0) & (d0 < 2.0))
        c_ind = ((d1 > -2.0) & (d1 <= -1.0),
                 (d1 > -1.0) & (d1 < 1.0),
                 (d1 >= 1.0) & (d1 < 2.0))

        Hs = []
        for ai in range(3):
            for bi in range(3):
                M = (r_ind[ai] & c_ind[bi]).astype(jnp.float32)
                Hab = jnp.dot(M, hm, preferred_element_type=jnp.float32)
                if ai == 1 and bi == 1:
                    Hab = Hab - hm
                Hs.append(Hab)
        Hcat = jnp.concatenate(Hs, axis=1)

        epre = jnp.dot(Hcat, WsocT_ref[...], preferred_element_type=jnp.float32)
        e = jax.nn.relu(epre * mb + bsoc_ref[...])

        r = jax.nn.relu(jnp.dot(xc, WinT_ref[...],
                                preferred_element_type=jnp.float32)
                        + bin_ref[...])

        concat = jnp.concatenate([r, e], axis=1)
        gates = ((jnp.dot(concat, WihT_ref[...], preferred_element_type=jnp.float32)
                  + bih_ref[...])
                 + jnp.dot(h, Whh_ref[...], preferred_element_type=jnp.float32)
                 + bhh_ref[...])
        gi = gates[:, 0 * HIDDEN:1 * HIDDEN]
        gf = gates[:, 1 * HIDDEN:2 * HIDDEN]
        gg = gates[:, 2 * HIDDEN:3 * HIDDEN]
        go = gates[:, 3 * HIDDEN:4 * HIDDEN]
        c2 = jax.nn.sigmoid(gf) * c + jax.nn.sigmoid(gi) * jnp.tanh(gg)
        h2 = jax.nn.sigmoid(go) * jnp.tanh(c2)

        o = (jnp.dot(h2, Wout_ref[...], preferred_element_type=jnp.float32)
             + bout_ref[...]) * mcol

        i3 = jnp.maximum(t - 3, 0)
        m3 = MC_ref[pl.ds(i3, 1), :, :].reshape(N, 1)
        cond = (mcol != 0.0) & (m3 == 0.0) & (t > 3)
        yv = Y_ref[pl.ds(t, 1), :, :].reshape(N, OUT_DIM)
        o = jnp.where(cond, yv, o)

        out_ref[pl.ds(t, 1), :, :] = o.reshape(1, N, OUT_DIM)
        return (h2, c2)

    ub = jnp.minimum(jnp.maximum(tpred + 1, 0), T)
    jax.lax.fori_loop(0, ub, frame, (h0_ref[...], c0_ref[...]))


def kernel(X, part_masks, all_h_t, all_c_t, Y, W_in, b_in, W_soc, b_soc,
           W_ih, W_hh, b_ih, b_hh, W_out, b_out, T_obs, T_pred):
    C0 = X[:, :, 2]
    C1 = X[:, :, 3]
    MC = part_masks[:, 0, :, None]

    blocks = []
    for a in (-1, 0, 1):
        for b in (-1, 0, 1):
            cell = (a + 2) * NSIZE + (b + 2)
            blocks.append(W_soc[:, cell * HIDDEN:(cell + 1) * HIDDEN].T)
    WsocT = jnp.concatenate(blocks, axis=0)
    WinT = W_in.T
    WihT = W_ih.T
    Whh = W_hh.T
    Wout = W_out.T
    tp = jnp.asarray(T_pred, jnp.int32).reshape(1, 1)

    in_specs = [pl.BlockSpec(memory_space=pltpu.VMEM)] * 17 + [
        pl.BlockSpec(memory_space=pltpu.SMEM)]

    return pl.pallas_call(
        _social_lstm_body,
        out_shape=jax.ShapeDtypeStruct((T, N, OUT_DIM), jnp.float32),
        in_specs=in_specs,
        out_specs=pl.BlockSpec(memory_space=pltpu.VMEM),
    )(X, C0, C1, MC, Y, all_h_t, all_c_t,
      WinT, b_in[None, :], WsocT, b_soc[None, :],
      WihT, Whh, b_ih[None, :], b_hh[None, :],
      Wout, b_out[None, :], tp)


# speedup vs baseline: 1.3723x; 1.1130x over previous
import jax
import jax.numpy as jnp
from jax.experimental import pallas as pl
from jax.experimental.pallas import tpu as pltpu

T = 16
N = 512
HIDDEN = 64
MEDIATE = 32
SOCIAL = 128
OUT_DIM = 2
NSIZE = 4
GRID = 1.0

_BINS = (-1.0, 0.0, 1.0)


def _social_lstm_body(X_ref, C0_ref, C1_ref, MC_ref, Y_ref, h0_ref, c0_ref,
                      WinT_ref, bin_ref, WsocT_ref, bsoc_ref,
                      WihT_ref, Whh_ref, bih_ref, bhh_ref,
                      Wout_ref, bout_ref, tp_ref, out_ref):
    tpred = tp_ref[0, 0]
    out_ref[...] = jnp.zeros((T, N, OUT_DIM), jnp.float32)

    def frame(t, carry):
        h, c = carry
        mcol = MC_ref[pl.ds(t, 1), :, :].reshape(N, 1)
        mb = (mcol != 0.0).astype(jnp.float32)
        hm = h * mb

        xc = X_ref[pl.ds(t, 1), :, 2:4].reshape(N, 2)
        x0c = xc[:, 0:1]
        x1c = xc[:, 1:2]
        x0r = C0_ref[pl.ds(t, 1), :]
        x1r = C1_ref[pl.ds(t, 1), :]
        d0 = x0r - x0c
        d1 = x1r - x1c

        g0 = jnp.trunc(d0)
        g1 = jnp.trunc(d1)
        r_ind = (g0 == -1.0, g0 == 0.0, g0 == 1.0)
        c_ind = (g1 == -1.0, g1 == 0.0, g1 == 1.0)

        Hs = []
        for ai in range(3):
            for bi in range(3):
                M = (r_ind[ai] & c_ind[bi]).astype(jnp.float32)
                Hab = jnp.dot(M, hm, preferred_element_type=jnp.float32)
                if ai == 1 and bi == 1:
                    Hab = Hab - hm
                Hs.append(Hab)
        Hcat = jnp.concatenate(Hs, axis=1)

        epre = jnp.dot(Hcat, WsocT_ref[...], preferred_element_type=jnp.float32)
        e = jax.nn.relu(epre * mb + bsoc_ref[...])

        r = jax.nn.relu(jnp.dot(xc, WinT_ref[...],
                                preferred_element_type=jnp.float32)
                        + bin_ref[...])

        concat = jnp.concatenate([r, e], axis=1)
        gates = ((jnp.dot(concat, WihT_ref[...], preferred_element_type=jnp.float32)
                  + bih_ref[...])
                 + jnp.dot(h, Whh_ref[...], preferred_element_type=jnp.float32)
                 + bhh_ref[...])
        gi = gates[:, 0 * HIDDEN:1 * HIDDEN]
        gf = gates[:, 1 * HIDDEN:2 * HIDDEN]
        gg = gates[:, 2 * HIDDEN:3 * HIDDEN]
        go = gates[:, 3 * HIDDEN:4 * HIDDEN]
        c2 = jax.nn.sigmoid(gf) * c + jax.nn.sigmoid(gi) * jnp.tanh(gg)
        h2 = jax.nn.sigmoid(go) * jnp.tanh(c2)

        o = (jnp.dot(h2, Wout_ref[...], preferred_element_type=jnp.float32)
             + bout_ref[...]) * mcol

        i3 = jnp.maximum(t - 3, 0)
        m3 = MC_ref[pl.ds(i3, 1), :, :].reshape(N, 1)
        cond = (mcol != 0.0) & (m3 == 0.0) & (t > 3)
        yv = Y_ref[pl.ds(t, 1), :, :].reshape(N, OUT_DIM)
        o = jnp.where(cond, yv, o)

        out_ref[pl.ds(t, 1), :, :] = o.reshape(1, N, OUT_DIM)
        return (h2, c2)

    ub = jnp.minimum(jnp.maximum(tpred + 1, 0), T)
    jax.lax.fori_loop(0, ub, frame, (h0_ref[...], c0_ref[...]))


def kernel(X, part_masks, all_h_t, all_c_t, Y, W_in, b_in, W_soc, b_soc,
           W_ih, W_hh, b_ih, b_hh, W_out, b_out, T_obs, T_pred):
    C0 = X[:, :, 2]
    C1 = X[:, :, 3]
    MC = part_masks[:, 0, :, None]

    blocks = []
    for a in (-1, 0, 1):
        for b in (-1, 0, 1):
            cell = (a + 2) * NSIZE + (b + 2)
            blocks.append(W_soc[:, cell * HIDDEN:(cell + 1) * HIDDEN].T)
    WsocT = jnp.concatenate(blocks, axis=0)
    WinT = W_in.T
    WihT = W_ih.T
    Whh = W_hh.T
    Wout = W_out.T
    tp = jnp.asarray(T_pred, jnp.int32).reshape(1, 1)

    in_specs = [pl.BlockSpec(memory_space=pltpu.VMEM)] * 17 + [
        pl.BlockSpec(memory_space=pltpu.SMEM)]

    return pl.pallas_call(
        _social_lstm_body,
        out_shape=jax.ShapeDtypeStruct((T, N, OUT_DIM), jnp.float32),
        in_specs=in_specs,
        out_specs=pl.BlockSpec(memory_space=pltpu.VMEM),
    )(X, C0, C1, MC, Y, all_h_t, all_c_t,
      WinT, b_in[None, :], WsocT, b_soc[None, :],
      WihT, Whh, b_ih[None, :], b_hh[None, :],
      Wout, b_out[None, :], tp)
